# baseline (speedup 1.0000x reference)
import functools

import jax
import jax.numpy as jnp
from jax import lax
from jax.experimental import pallas as pl
from jax.experimental.pallas import tpu as pltpu
from jax.experimental.pallas import tpu_sc as plsc

BATCH = 16384
EMB = 64
IDX_CHUNK = 128


def _mf_body(nw, bpw, u_id, i_id, user_emb, user_bias, item_emb, item_bias,
             mean, out, uidx, iidx, u_rows, i_rows, bu, bi, mean_v, out_v, sem):
    c = lax.axis_index("c")
    s = lax.axis_index("s")
    wid = s * 2 + c
    base = wid * bpw
    n_chunks = bpw // IDX_CHUNK

    for j in range(n_chunks):
        pltpu.sync_copy(u_id.at[pl.ds(base + j * IDX_CHUNK, IDX_CHUNK)], uidx.at[j])
        pltpu.sync_copy(i_id.at[pl.ds(base + j * IDX_CHUNK, IDX_CHUNK)], iidx.at[j])
    pltpu.sync_copy(mean, mean_v.at[pl.ds(0, 1)])

    copies = []
    for j in range(n_chunks):
        sl = pl.ds(j * IDX_CHUNK, IDX_CHUNK)
        copies.append(pltpu.async_copy(user_emb.at[uidx.at[j]], u_rows.at[sl], sem))
        copies.append(pltpu.async_copy(item_emb.at[iidx.at[j]], i_rows.at[sl], sem))
        copies.append(pltpu.async_copy(user_bias.at[uidx.at[j]], bu.at[sl], sem))
        copies.append(pltpu.async_copy(item_bias.at[iidx.at[j]], bi.at[sl], sem))
    for cp in copies:
        cp.wait()

    mean_s = mean_v[pl.ds(0, 16)][0]
    zeros = jnp.zeros((16,), jnp.int32)

    def group(g, carry):
        rowv = g * 16 + lax.iota(jnp.int32, 16)
        b_u = bu[pl.ds(g * 16, 16)]
        b_i = bi[pl.ds(g * 16, 16)]
        acc = b_u + b_i + mean_s
        for k in range(EMB):
            colv = jnp.full((16,), k, jnp.int32)
            uu = plsc.load_gather(u_rows, [rowv, colv])
            ii = plsc.load_gather(i_rows, [rowv, colv])
            acc = acc + uu * ii
        out_v[pl.ds(g * 16, 16)] = acc
        return carry

    lax.fori_loop(0, bpw // 16, group, 0)
    pltpu.sync_copy(out_v, out.at[pl.ds(base, bpw)])


def kernel(u_id, i_id, user_emb, user_bias, item_emb, item_bias, mean):
    info = plsc.get_sparse_core_info()
    nw = info.num_cores * info.num_subcores
    bpw = BATCH // nw
    mesh = plsc.VectorSubcoreMesh(core_axis_name="c", subcore_axis_name="s")

    k = functools.partial(
        pl.kernel,
        mesh=mesh,
        out_type=jax.ShapeDtypeStruct((BATCH,), jnp.float32),
        compiler_params=pltpu.CompilerParams(needs_layout_passes=False,
                                             use_tc_tiling_on_sc=False),
        scratch_types=[
            pltpu.VMEM((bpw // IDX_CHUNK, IDX_CHUNK), jnp.int32),
            pltpu.VMEM((bpw // IDX_CHUNK, IDX_CHUNK), jnp.int32),
            pltpu.VMEM((bpw, EMB), jnp.float32),
            pltpu.VMEM((bpw, EMB), jnp.float32),
            pltpu.VMEM((bpw,), jnp.float32),
            pltpu.VMEM((bpw,), jnp.float32),
            pltpu.VMEM((16,), jnp.float32),
            pltpu.VMEM((bpw,), jnp.float32),
            pltpu.SemaphoreType.DMA,
        ],
    )(functools.partial(_mf_body, nw, bpw))

    return k(u_id.astype(jnp.int32), i_id.astype(jnp.int32),
             user_emb, user_bias.reshape(-1), item_emb, item_bias.reshape(-1),
             mean)


# speedup vs baseline: 2.0211x; 2.0211x over previous
import functools

import jax
import jax.numpy as jnp
from jax import lax
from jax.experimental import pallas as pl
from jax.experimental.pallas import tpu as pltpu
from jax.experimental.pallas import tpu_sc as plsc

BATCH = 16384
EMB = 64
IDX_CHUNK = 128
WIN = 128
SB = 4


def _mf_body(nw, bpw, u_id, i_id, uT, ub, iT, ib, mean, out,
             uidx, iidx, u_win, i_win, psum, bu, bi, mean_v, out_v, sem, bsem):
    c = lax.axis_index("c")
    s = lax.axis_index("s")
    wid = s * 2 + c
    base = wid * bpw
    n_chunks = bpw // IDX_CHUNK

    for j in range(n_chunks):
        pltpu.sync_copy(u_id.at[pl.ds(base + j * IDX_CHUNK, IDX_CHUNK)], uidx.at[j])
        pltpu.sync_copy(i_id.at[pl.ds(base + j * IDX_CHUNK, IDX_CHUNK)], iidx.at[j])
    pltpu.sync_copy(mean, mean_v.at[pl.ds(0, 1)])

    bias_copies = []
    for j in range(n_chunks):
        sl = pl.ds(j * IDX_CHUNK, IDX_CHUNK)
        bias_copies.append(pltpu.async_copy(ub.at[uidx.at[j]], bu.at[sl], bsem))
        bias_copies.append(pltpu.async_copy(ib.at[iidx.at[j]], bi.at[sl], bsem))
    for cp in bias_copies:
        cp.wait()

    mean_s = mean_v[pl.ds(0, 16)][0]
    iota16 = lax.iota(jnp.int32, 16)

    def group(g, carry):
        uvec = uidx.at[g // 8][pl.ds((g % 8) * 16, 16)]
        ivec = iidx.at[g // 8][pl.ds((g % 8) * 16, 16)]
        for sb in range(16 // SB):
            copies = []
            lanes = []
            for j in range(SB):
                l = sb * SB + j
                cu = uvec[l]
                ci = ivec[l]
                ua = pl.multiple_of((cu // WIN) * WIN, WIN)
                ia = pl.multiple_of((ci // WIN) * WIN, WIN)
                lanes.append((cu - ua, ci - ia))
                copies.append(pltpu.async_copy(
                    uT.at[:, pl.ds(ua, WIN)], u_win.at[j], sem))
                copies.append(pltpu.async_copy(
                    iT.at[:, pl.ds(ia, WIN)], i_win.at[j], sem))
            for cp in copies:
                cp.wait()
            for j in range(SB):
                l = sb * SB + j
                lane_u, lane_i = lanes[j]
                lu = jnp.full((16,), 0, jnp.int32) + lane_u
                li = jnp.full((16,), 0, jnp.int32) + lane_i
                acc = None
                for fb in range(EMB // 16):
                    fvec = fb * 16 + iota16
                    uu = plsc.load_gather(u_win.at[j], [fvec, lu])
                    ii = plsc.load_gather(i_win.at[j], [fvec, li])
                    p = uu * ii
                    acc = p if acc is None else acc + p
                psum[l, pl.ds(0, 16)] = acc
        red = None
        for jj in range(16):
            col = plsc.load_gather(psum, [iota16, jnp.full((16,), jj, jnp.int32)])
            red = col if red is None else red + col
        o = g * 16
        out_v[pl.ds(o, 16)] = red + bu[pl.ds(o, 16)] + bi[pl.ds(o, 16)] + mean_s
        return carry

    lax.fori_loop(0, bpw // 16, group, 0)
    pltpu.sync_copy(out_v, out.at[pl.ds(base, bpw)])


def kernel(u_id, i_id, user_emb, user_bias, item_emb, item_bias, mean):
    info = plsc.get_sparse_core_info()
    nw = info.num_cores * info.num_subcores
    bpw = BATCH // nw
    mesh = plsc.VectorSubcoreMesh(core_axis_name="c", subcore_axis_name="s")

    k = functools.partial(
        pl.kernel,
        mesh=mesh,
        out_type=jax.ShapeDtypeStruct((BATCH,), jnp.float32),
        compiler_params=pltpu.CompilerParams(needs_layout_passes=False,
                                             use_tc_tiling_on_sc=True),
        scratch_types=[
            pltpu.VMEM((bpw // IDX_CHUNK, IDX_CHUNK), jnp.int32),
            pltpu.VMEM((bpw // IDX_CHUNK, IDX_CHUNK), jnp.int32),
            pltpu.VMEM((SB, EMB, WIN), jnp.float32),
            pltpu.VMEM((SB, EMB, WIN), jnp.float32),
            pltpu.VMEM((16, 16), jnp.float32),
            pltpu.VMEM((bpw,), jnp.float32),
            pltpu.VMEM((bpw,), jnp.float32),
            pltpu.VMEM((16,), jnp.float32),
            pltpu.VMEM((bpw,), jnp.float32),
            pltpu.SemaphoreType.DMA,
            pltpu.SemaphoreType.DMA,
        ],
    )(functools.partial(_mf_body, nw, bpw))

    return k(u_id.astype(jnp.int32), i_id.astype(jnp.int32),
             user_emb.T, user_bias.reshape(-1), item_emb.T,
             item_bias.reshape(-1), mean)


# speedup vs baseline: 2.2452x; 1.1109x over previous
import functools

import jax
import jax.numpy as jnp
from jax import lax
from jax.experimental import pallas as pl
from jax.experimental.pallas import tpu as pltpu
from jax.experimental.pallas import tpu_sc as plsc

BATCH = 16384
EMB = 64
IDX_CHUNK = 128
WIN = 128
NBUF = 4
LAG = 3


def _win_start(col):
    return pl.multiple_of((col // WIN) * WIN, WIN)


def _mf_body(nw, bpw, u_id, i_id, uT, ub, iT, ib, mean, out,
             uidx, iidx, uflat, iflat, u_win, i_win, psum, bu, bi, mean_v,
             out_v, sem0, sem1, sem2, sem3, bsem):
    c = lax.axis_index("c")
    s = lax.axis_index("s")
    wid = s * 2 + c
    base = wid * bpw
    n_chunks = bpw // IDX_CHUNK
    sems = [sem0, sem1, sem2, sem3]

    pltpu.sync_copy(u_id.at[pl.ds(base, bpw)], uflat.at[pl.ds(0, bpw)])
    pltpu.sync_copy(i_id.at[pl.ds(base, bpw)], iflat.at[pl.ds(0, bpw)])
    for j in range(n_chunks):
        pltpu.sync_copy(u_id.at[pl.ds(base + j * IDX_CHUNK, IDX_CHUNK)], uidx.at[j])
        pltpu.sync_copy(i_id.at[pl.ds(base + j * IDX_CHUNK, IDX_CHUNK)], iidx.at[j])
    pltpu.sync_copy(mean, mean_v.at[pl.ds(0, 1)])

    bias_copies = []
    for j in range(n_chunks):
        sl = pl.ds(j * IDX_CHUNK, IDX_CHUNK)
        bias_copies.append(pltpu.async_copy(ub.at[uidx.at[j]], bu.at[sl], bsem))
        bias_copies.append(pltpu.async_copy(ib.at[iidx.at[j]], bi.at[sl], bsem))
    for cp in bias_copies:
        cp.wait()

    mean_s = mean_v[pl.ds(0, 16)][0]
    iota16 = lax.iota(jnp.int32, 16)
    n_groups = bpw // 16

    def extract(win_ref, slot, lane):
        lv = jnp.full((16,), 0, jnp.int32) + lane
        vals = []
        for fb in range(EMB // 16):
            vals.append(plsc.load_gather(win_ref.at[slot], [fb * 16 + iota16, lv]))
        return vals

    def group_body(g, carry):
        cur_u = uflat[pl.ds(g * 16, 16)]
        cur_i = iflat[pl.ds(g * 16, 16)]
        po = jnp.maximum(g - 1, 0) * 16
        prev_u = uflat[pl.ds(po, 16)]
        prev_i = iflat[pl.ds(po, 16)]
        for j in range(16):
            slot = j % NBUF
            @pl.when(g < n_groups)
            def _fire():
                ua = _win_start(cur_u[j])
                ia = _win_start(cur_i[j])
                pltpu.async_copy(uT.at[:, pl.ds(ua, WIN)], u_win.at[slot], sems[slot])
                pltpu.async_copy(iT.at[:, pl.ds(ia, WIN)], i_win.at[slot], sems[slot])

            qj = j - LAG
            qslot = qj % NBUF
            if j >= LAG:
                cond = g < n_groups
                cu, ci = cur_u[qj], cur_i[qj]
            else:
                cond = g >= 1
                cu, ci = prev_u[qj + 16], prev_i[qj + 16]

            @pl.when(cond)
            def _drain():
                pltpu.make_async_copy(uT.at[:, pl.ds(0, WIN)],
                                      u_win.at[qslot], sems[qslot]).wait()
                pltpu.make_async_copy(iT.at[:, pl.ds(0, WIN)],
                                      i_win.at[qslot], sems[qslot]).wait()
                lane_u = cu - _win_start(cu)
                lane_i = ci - _win_start(ci)
                us = extract(u_win, qslot, lane_u)
                vs = extract(i_win, qslot, lane_i)
                acc = None
                for uu, ii in zip(us, vs):
                    p = uu * ii
                    acc = p if acc is None else acc + p
                row = qj % 16
                psum[pl.ds(row * 16, 16)] = acc

            if j == LAG - 1:
                @pl.when(g >= 1)
                def _reduce():
                    red = None
                    for jj in range(16):
                        col = plsc.load_gather(
                            psum, [iota16 * 16 + jj])
                        red = col if red is None else red + col
                    o = (g - 1) * 16
                    out_v[pl.ds(o, 16)] = (red + bu[pl.ds(o, 16)]
                                           + bi[pl.ds(o, 16)] + mean_s)
        return carry

    lax.fori_loop(0, n_groups + 1, group_body, 0)
    pltpu.sync_copy(out_v, out.at[pl.ds(base, bpw)])


def kernel(u_id, i_id, user_emb, user_bias, item_emb, item_bias, mean):
    info = plsc.get_sparse_core_info()
    nw = info.num_cores * info.num_subcores
    bpw = BATCH // nw
    mesh = plsc.VectorSubcoreMesh(core_axis_name="c", subcore_axis_name="s")

    k = functools.partial(
        pl.kernel,
        mesh=mesh,
        out_type=jax.ShapeDtypeStruct((BATCH,), jnp.float32),
        compiler_params=pltpu.CompilerParams(needs_layout_passes=False,
                                             use_tc_tiling_on_sc=True),
        scratch_types=[
            pltpu.VMEM((bpw // IDX_CHUNK, IDX_CHUNK), jnp.int32),
            pltpu.VMEM((bpw // IDX_CHUNK, IDX_CHUNK), jnp.int32),
            pltpu.VMEM((bpw + 32,), jnp.int32),
            pltpu.VMEM((bpw + 32,), jnp.int32),
            pltpu.VMEM((NBUF, EMB, WIN), jnp.float32),
            pltpu.VMEM((NBUF, EMB, WIN), jnp.float32),
            pltpu.VMEM((256,), jnp.float32),
            pltpu.VMEM((bpw,), jnp.float32),
            pltpu.VMEM((bpw,), jnp.float32),
            pltpu.VMEM((16,), jnp.float32),
            pltpu.VMEM((bpw,), jnp.float32),
            pltpu.SemaphoreType.DMA,
            pltpu.SemaphoreType.DMA,
            pltpu.SemaphoreType.DMA,
            pltpu.SemaphoreType.DMA,
            pltpu.SemaphoreType.DMA,
        ],
    )(functools.partial(_mf_body, nw, bpw))

    return k(u_id.astype(jnp.int32), i_id.astype(jnp.int32),
             user_emb.T, user_bias.reshape(-1), item_emb.T,
             item_bias.reshape(-1), mean)
